# baseline (speedup 1.0000x reference)
import functools

import jax
import jax.numpy as jnp
from jax import lax
from jax.experimental import pallas as pl
from jax.experimental.pallas import tpu as pltpu
from jax.experimental.pallas import tpu_sc as plsc

BATCH = 128
SIZE = 32768
N_GROUPS = 8
GROUP_SIZE = SIZE // N_GROUPS
THRESHOLD = 1.0

NUM_CORES = 2
NUM_SUBCORES = 16
NUM_WORKERS = NUM_CORES * NUM_SUBCORES
ROW_BLOCKS = NUM_WORKERS // N_GROUPS
ROWS_PER_WORKER = BATCH // ROW_BLOCKS
LANES = 16
NBUF = 4

_mesh = plsc.VectorSubcoreMesh(
    core_axis_name="c", subcore_axis_name="s",
    num_cores=NUM_CORES, num_subcores=NUM_SUBCORES)

_row_f32 = pltpu.VMEM((GROUP_SIZE,), jnp.float32)


@functools.partial(
    pl.kernel,
    out_type=(
        jax.ShapeDtypeStruct((BATCH, SIZE), jnp.float32),
        jax.ShapeDtypeStruct((BATCH, SIZE), jnp.float32),
    ),
    mesh=_mesh,
    compiler_params=pltpu.CompilerParams(needs_layout_passes=False),
    scratch_types=(
        [_row_f32]
        + [_row_f32] * NBUF
        + [_row_f32] * NBUF
        + [_row_f32] * NBUF
        + [_row_f32] * NBUF
        + [pltpu.SemaphoreType.DMA] * NBUF
        + [pltpu.SemaphoreType.DMA] * NBUF
    ),
)
def _wta_sc(i_hbm, v_hbm, beta_hbm, vout_hbm, sout_hbm, b_buf, *rest):
    i_bufs = rest[0:NBUF]
    v_bufs = rest[NBUF:2 * NBUF]
    vo_bufs = rest[2 * NBUF:3 * NBUF]
    s_bufs = rest[3 * NBUF:4 * NBUF]
    in_sems = rest[4 * NBUF:5 * NBUF]
    out_sems = rest[5 * NBUF:6 * NBUF]

    wid = lax.axis_index("s") * NUM_CORES + lax.axis_index("c")
    g = wid % N_GROUPS
    row0 = (wid // N_GROUPS) * ROWS_PER_WORKER
    col0 = g * GROUP_SIZE

    pltpu.sync_copy(beta_hbm.at[pl.ds(col0, GROUP_SIZE)], b_buf)

    def hbm_slice(ref, r):
        return ref.at[row0 + r, pl.ds(col0, GROUP_SIZE)]

    def issue_in(r, b):
        pltpu.async_copy(hbm_slice(i_hbm, r), i_bufs[b], in_sems[b])
        pltpu.async_copy(hbm_slice(v_hbm, r), v_bufs[b], in_sems[b])

    def wait_in(b):
        pltpu.make_async_copy(hbm_slice(i_hbm, 0), i_bufs[b], in_sems[b]).wait()
        pltpu.make_async_copy(hbm_slice(v_hbm, 0), v_bufs[b], in_sems[b]).wait()

    def issue_out(r, b):
        pltpu.async_copy(vo_bufs[b], hbm_slice(vout_hbm, r), out_sems[b])
        pltpu.async_copy(s_bufs[b], hbm_slice(sout_hbm, r), out_sems[b])

    def wait_out(b):
        pltpu.make_async_copy(vo_bufs[b], hbm_slice(vout_hbm, 0), out_sems[b]).wait()
        pltpu.make_async_copy(s_bufs[b], hbm_slice(sout_hbm, 0), out_sems[b]).wait()

    issue_in(0, 0)
    issue_in(1, 1)

    def step(p, b0, b1):
        r = p * 2

        @pl.when(r + 2 < ROWS_PER_WORKER)
        def _():
            issue_in(r + 2, (b0 + 2) % NBUF)
            issue_in(r + 3, (b1 + 2) % NBUF)

        wait_in(b0)
        wait_in(b1)

        @pl.when(p >= 2)
        def _():
            wait_out(b0)
            wait_out(b1)

        zero = jnp.zeros((LANES,), jnp.float32)

        @plsc.parallel_loop(0, GROUP_SIZE, LANES, unroll=8,
                            carry=(zero, zero))
        def ms(o, mc):
            m0, m1 = mc
            sl = pl.ds(o, LANES)
            bb = b_buf[sl]
            v0 = bb * v_bufs[b0][sl] + i_bufs[b0][sl]
            v1 = bb * v_bufs[b1][sl] + i_bufs[b1][sl]
            spk0 = jnp.where(v0 >= THRESHOLD, 1.0, 0.0)
            spk1 = jnp.where(v1 >= THRESHOLD, 1.0, 0.0)
            vo_bufs[b0][sl] = v0 - spk0
            vo_bufs[b1][sl] = v1 - spk1
            s_bufs[b0][sl] = spk0
            s_bufs[b1][sl] = spk1
            return (jnp.maximum(m0, spk0), jnp.maximum(m1, spk1))

        for (mrow, b) in ((ms[0], b0), (ms[1], b1)):
            c = plsc.cummax(mrow)
            gm = plsc.cummax(jnp.maximum(c, lax.rev(c, (0,))))

            @plsc.parallel_loop(0, GROUP_SIZE, LANES, unroll=8)
            def _(o):
                sl = pl.ds(o, LANES)
                s = s_bufs[b][sl]
                s_bufs[b][sl] = jnp.where(s >= gm, s, 0.0)

        issue_out(r, b0)
        issue_out(r + 1, b1)

    def outer(t, carry):
        step(t * 2, 0, 1)
        step(t * 2 + 1, 2, 3)
        return carry

    lax.fori_loop(0, ROWS_PER_WORKER // NBUF, outer, 0)
    for b in range(NBUF):
        wait_out(b)


def kernel(input_current, v_prev, beta):
    return _wta_sc(input_current, v_prev, beta)


# speedup vs baseline: 1.0252x; 1.0252x over previous
import functools

import jax
import jax.numpy as jnp
from jax import lax
from jax.experimental import pallas as pl
from jax.experimental.pallas import tpu as pltpu
from jax.experimental.pallas import tpu_sc as plsc

BATCH = 128
SIZE = 32768
N_GROUPS = 8
GROUP_SIZE = SIZE // N_GROUPS
THRESHOLD = 1.0

NUM_CORES = 2
NUM_SUBCORES = 16
NUM_WORKERS = NUM_CORES * NUM_SUBCORES
ROW_BLOCKS = NUM_WORKERS // N_GROUPS
ROWS_PER_WORKER = BATCH // ROW_BLOCKS
LANES = 16
NBUF = 4

_mesh = plsc.VectorSubcoreMesh(
    core_axis_name="c", subcore_axis_name="s",
    num_cores=NUM_CORES, num_subcores=NUM_SUBCORES)

_row_f32 = pltpu.VMEM((GROUP_SIZE,), jnp.float32)


@functools.partial(
    pl.kernel,
    out_type=(
        jax.ShapeDtypeStruct((BATCH, SIZE), jnp.float32),
        jax.ShapeDtypeStruct((BATCH, SIZE), jnp.float32),
    ),
    mesh=_mesh,
    compiler_params=pltpu.CompilerParams(needs_layout_passes=False),
    scratch_types=(
        [_row_f32]
        + [_row_f32] * NBUF
        + [_row_f32] * NBUF
        + [_row_f32] * NBUF
        + [_row_f32] * NBUF
        + [pltpu.SemaphoreType.DMA] * NBUF
        + [pltpu.SemaphoreType.DMA] * NBUF
    ),
)
def _wta_sc(i_hbm, v_hbm, beta_hbm, vout_hbm, sout_hbm, b_buf, *rest):
    i_bufs = rest[0:NBUF]
    v_bufs = rest[NBUF:2 * NBUF]
    vo_bufs = rest[2 * NBUF:3 * NBUF]
    s_bufs = rest[3 * NBUF:4 * NBUF]
    in_sems = rest[4 * NBUF:5 * NBUF]
    out_sems = rest[5 * NBUF:6 * NBUF]

    wid = lax.axis_index("s") * NUM_CORES + lax.axis_index("c")
    g = wid % N_GROUPS
    row0 = (wid // N_GROUPS) * ROWS_PER_WORKER
    col0 = g * GROUP_SIZE

    pltpu.sync_copy(beta_hbm.at[pl.ds(col0, GROUP_SIZE)], b_buf)

    def hbm_slice(ref, r):
        return ref.at[row0 + r, pl.ds(col0, GROUP_SIZE)]

    def issue_in(r, b):
        pltpu.async_copy(hbm_slice(i_hbm, r), i_bufs[b], in_sems[b])
        pltpu.async_copy(hbm_slice(v_hbm, r), v_bufs[b], in_sems[b])

    def wait_in(b):
        pltpu.make_async_copy(hbm_slice(i_hbm, 0), i_bufs[b], in_sems[b]).wait()
        pltpu.make_async_copy(hbm_slice(v_hbm, 0), v_bufs[b], in_sems[b]).wait()

    def issue_out(r, b):
        pltpu.async_copy(vo_bufs[b], hbm_slice(vout_hbm, r), out_sems[b])
        pltpu.async_copy(s_bufs[b], hbm_slice(sout_hbm, r), out_sems[b])

    def wait_out(b):
        pltpu.make_async_copy(vo_bufs[b], hbm_slice(vout_hbm, 0), out_sems[b]).wait()
        pltpu.make_async_copy(s_bufs[b], hbm_slice(sout_hbm, 0), out_sems[b]).wait()

    for b in range(NBUF - 1):
        issue_in(b, b)

    def step(r, b):
        @pl.when(r + NBUF - 1 < ROWS_PER_WORKER)
        def _():
            issue_in(r + NBUF - 1, (b + NBUF - 1) % NBUF)

        wait_in(b)

        @pl.when(r >= NBUF)
        def _():
            wait_out(b)

        @plsc.parallel_loop(0, GROUP_SIZE, LANES, unroll=8,
                            carry=jnp.zeros((LANES,), jnp.float32))
        def m(o, mc):
            sl = pl.ds(o, LANES)
            v = b_buf[sl] * v_bufs[b][sl] + i_bufs[b][sl]
            spk = jnp.where(v >= THRESHOLD, 1.0, 0.0)
            vo_bufs[b][sl] = v - spk
            s_bufs[b][sl] = spk
            return jnp.maximum(mc, spk)

        c = plsc.cummax(m)
        gm = plsc.cummax(jnp.maximum(c, lax.rev(c, (0,))))

        @plsc.parallel_loop(0, GROUP_SIZE, LANES, unroll=8)
        def _(o):
            sl = pl.ds(o, LANES)
            s = s_bufs[b][sl]
            s_bufs[b][sl] = jnp.where(s >= gm, s, 0.0)

        issue_out(r, b)

    def outer(t, carry):
        for b in range(NBUF):
            step(t * NBUF + b, b)
        return carry

    lax.fori_loop(0, ROWS_PER_WORKER // NBUF, outer, 0)
    for b in range(NBUF):
        wait_out(b)


def kernel(input_current, v_prev, beta):
    return _wta_sc(input_current, v_prev, beta)


# speedup vs baseline: 1.0261x; 1.0009x over previous
import functools

import jax
import jax.numpy as jnp
from jax import lax
from jax.experimental import pallas as pl
from jax.experimental.pallas import tpu as pltpu
from jax.experimental.pallas import tpu_sc as plsc

BATCH = 128
SIZE = 32768
N_GROUPS = 8
GROUP_SIZE = SIZE // N_GROUPS
THRESHOLD = 1.0

NUM_CORES = 2
NUM_SUBCORES = 16
NUM_WORKERS = NUM_CORES * NUM_SUBCORES
ROW_BLOCKS = NUM_WORKERS // N_GROUPS
ROWS_PER_WORKER = BATCH // ROW_BLOCKS
LANES = 16
NBUF = 4

_mesh = plsc.VectorSubcoreMesh(
    core_axis_name="c", subcore_axis_name="s",
    num_cores=NUM_CORES, num_subcores=NUM_SUBCORES)

_row_f32 = pltpu.VMEM((GROUP_SIZE,), jnp.float32)


@functools.partial(
    pl.kernel,
    out_type=(
        jax.ShapeDtypeStruct((BATCH, SIZE), jnp.float32),
        jax.ShapeDtypeStruct((BATCH, SIZE), jnp.float32),
    ),
    mesh=_mesh,
    compiler_params=pltpu.CompilerParams(needs_layout_passes=False),
    scratch_types=(
        [_row_f32]
        + [_row_f32] * NBUF
        + [_row_f32] * NBUF
        + [_row_f32] * NBUF
        + [_row_f32] * NBUF
        + [pltpu.SemaphoreType.DMA] * NBUF
        + [pltpu.SemaphoreType.DMA] * NBUF
    ),
)
def _wta_sc(i_hbm, v_hbm, beta_hbm, vout_hbm, sout_hbm, b_buf, *rest):
    i_bufs = rest[0:NBUF]
    v_bufs = rest[NBUF:2 * NBUF]
    vo_bufs = rest[2 * NBUF:3 * NBUF]
    s_bufs = rest[3 * NBUF:4 * NBUF]
    in_sems = rest[4 * NBUF:5 * NBUF]
    out_sems = rest[5 * NBUF:6 * NBUF]

    wid = lax.axis_index("c") * NUM_SUBCORES + lax.axis_index("s")
    g = wid // ROW_BLOCKS
    row0 = (wid % ROW_BLOCKS) * ROWS_PER_WORKER
    col0 = g * GROUP_SIZE

    pltpu.sync_copy(beta_hbm.at[pl.ds(col0, GROUP_SIZE)], b_buf)

    def hbm_slice(ref, r):
        return ref.at[row0 + r, pl.ds(col0, GROUP_SIZE)]

    def issue_in(r, b):
        pltpu.async_copy(hbm_slice(i_hbm, r), i_bufs[b], in_sems[b])
        pltpu.async_copy(hbm_slice(v_hbm, r), v_bufs[b], in_sems[b])

    def wait_in(b):
        pltpu.make_async_copy(hbm_slice(i_hbm, 0), i_bufs[b], in_sems[b]).wait()
        pltpu.make_async_copy(hbm_slice(v_hbm, 0), v_bufs[b], in_sems[b]).wait()

    def issue_out(r, b):
        pltpu.async_copy(vo_bufs[b], hbm_slice(vout_hbm, r), out_sems[b])
        pltpu.async_copy(s_bufs[b], hbm_slice(sout_hbm, r), out_sems[b])

    def wait_out(b):
        pltpu.make_async_copy(vo_bufs[b], hbm_slice(vout_hbm, 0), out_sems[b]).wait()
        pltpu.make_async_copy(s_bufs[b], hbm_slice(sout_hbm, 0), out_sems[b]).wait()

    for b in range(NBUF - 1):
        issue_in(b, b)

    def step(r, b):
        @pl.when(r + NBUF - 1 < ROWS_PER_WORKER)
        def _():
            issue_in(r + NBUF - 1, (b + NBUF - 1) % NBUF)

        wait_in(b)

        @pl.when(r >= NBUF)
        def _():
            wait_out(b)

        @plsc.parallel_loop(0, GROUP_SIZE, LANES, unroll=8,
                            carry=jnp.zeros((LANES,), jnp.float32))
        def m(o, mc):
            sl = pl.ds(o, LANES)
            v = b_buf[sl] * v_bufs[b][sl] + i_bufs[b][sl]
            spk = jnp.where(v >= THRESHOLD, 1.0, 0.0)
            vo_bufs[b][sl] = v - spk
            s_bufs[b][sl] = spk
            return jnp.maximum(mc, spk)

        c = plsc.cummax(m)
        gm = plsc.cummax(jnp.maximum(c, lax.rev(c, (0,))))

        @plsc.parallel_loop(0, GROUP_SIZE, LANES, unroll=8)
        def _(o):
            sl = pl.ds(o, LANES)
            s = s_bufs[b][sl]
            s_bufs[b][sl] = jnp.where(s >= gm, s, 0.0)

        issue_out(r, b)

    def outer(t, carry):
        for b in range(NBUF):
            step(t * NBUF + b, b)
        return carry

    lax.fori_loop(0, ROWS_PER_WORKER // NBUF, outer, 0)
    for b in range(NBUF):
        wait_out(b)


def kernel(input_current, v_prev, beta):
    return _wta_sc(input_current, v_prev, beta)
